# baseline (speedup 1.0000x reference)
import functools

import jax
import jax.numpy as jnp
from jax import lax
from jax.experimental import pallas as pl
from jax.experimental.pallas import tpu as pltpu
from jax.experimental.pallas import tpu_sc as plsc

_N = 10000
_E = 320000
_DIN = 128
_DOUT = 256
_NET = 4
_STEPS = 8

_NC = 2
_NS = 16
_CHUNK = 128
_EPS = _E // (_NC * _NS)
_NCHUNK = 80
_GROUPS = 5
_GPC = _NCHUNK // _GROUPS
_EPS_PAD = _NCHUNK * _CHUNK
_NROWS = 640
_N_PAD = _NROWS * _NS
_DUMMY = _N

_BN = 400
_NB = _N // _BN


def _sc_segsum_body(t_hbm, gidx_hbm, dst_hbm, zero_hbm, out_hbm,
                    gidx_v, dst_v, rows0_v, rows1_v, gsem, ssem, acc_sh):
    c = lax.axis_index("c")
    s = lax.axis_index("s")
    pltpu.sync_copy(zero_hbm, acc_sh.at[pl.ds(s * _NROWS, _NROWS)])
    plsc.subcore_barrier()

    bufs = [rows0_v, rows1_v]
    nb = len(bufs)
    for grp in range(_GROUPS):
        pltpu.sync_copy(gidx_hbm.at[c, s, pl.ds(grp * _GPC, _GPC)], gidx_v)
        pltpu.sync_copy(dst_hbm.at[c, s, pl.ds(grp * _GPC, _GPC)], dst_v)
        g_desc = [None] * _GPC
        s_desc = [None] * _GPC
        for j in range(nb - 1):
            g_desc[j] = pltpu.async_copy(
                t_hbm.at[gidx_v.at[j]], bufs[j], gsem)
        for j in range(_GPC):
            b = bufs[j % nb]
            if j + nb - 1 < _GPC:
                if j >= 1:
                    s_desc[j - 1].wait()
                g_desc[j + nb - 1] = pltpu.async_copy(
                    t_hbm.at[gidx_v.at[j + nb - 1]], bufs[(j + nb - 1) % nb],
                    gsem)
            g_desc[j].wait()
            s_desc[j] = pltpu.async_copy(
                b, acc_sh.at[dst_v.at[j]], ssem, add=True)
        for j in range(_GPC - nb, _GPC):
            s_desc[j].wait()
    plsc.subcore_barrier()
    pltpu.sync_copy(acc_sh.at[pl.ds(s * _NROWS, _NROWS)],
                    out_hbm.at[c, pl.ds(s * _NROWS, _NROWS)])


@functools.lru_cache(maxsize=None)
def _build_sc_segsum():
    return pl.kernel(
        _sc_segsum_body,
        out_type=jax.ShapeDtypeStruct((_NC, _N_PAD, 2, 128), jnp.bfloat16),
        mesh=plsc.VectorSubcoreMesh(core_axis_name="c", subcore_axis_name="s",
                                    num_cores=_NC, num_subcores=_NS),
        compiler_params=pltpu.CompilerParams(use_tc_tiling_on_sc=False),
        scratch_types=[
            pltpu.VMEM((_GPC, _CHUNK), jnp.int32),
            pltpu.VMEM((_GPC, _CHUNK), jnp.int32),
            pltpu.VMEM((_CHUNK, 2, 128), jnp.bfloat16),
            pltpu.VMEM((_CHUNK, 2, 128), jnp.bfloat16),
            pltpu.SemaphoreType.DMA,
            pltpu.SemaphoreType.DMA,
            pltpu.VMEM_SHARED((_N_PAD, 2, 128), jnp.bfloat16),
        ],
    )


def _prep_body(h_ref, w_ref, b_ref, t_ref):
    res = jnp.dot(h_ref[...].astype(jnp.bfloat16), w_ref[...],
                  preferred_element_type=jnp.float32) + b_ref[...]
    res = res.astype(jnp.bfloat16)
    for e in range(_NET):
        t_ref[e] = res[:, e * _DOUT:(e + 1) * _DOUT]


def _prep_call(h, wcat, bcat):
    return pl.pallas_call(
        _prep_body,
        grid=(_NB,),
        in_specs=[
            pl.BlockSpec((_BN, _DOUT), lambda i: (i, 0)),
            pl.BlockSpec((_DOUT, _NC * _NET * 128), lambda i: (0, 0)),
            pl.BlockSpec((1, _NC * _NET * 128), lambda i: (0, 0)),
        ],
        out_specs=pl.BlockSpec((_NET, _BN, _DOUT), lambda i: (0, i, 0)),
        out_shape=jax.ShapeDtypeStruct((_NET, _N, _DOUT), jnp.bfloat16),
    )(h, wcat, bcat)


def _gru_body(aa_ref, h_ref, wih_ref, whh_ref, bih_ref, bhh_ref, wcat_ref,
              bcat_ref, out_ref, t_ref):
    h = h_ref[...]
    a = (aa_ref[0] + aa_ref[1]).astype(jnp.bfloat16)
    gi = jnp.dot(a, wih_ref[...], preferred_element_type=jnp.float32) \
        + bih_ref[...]
    gh = jnp.dot(h.astype(jnp.bfloat16), whh_ref[...],
                 preferred_element_type=jnp.float32) + bhh_ref[...]
    r = jax.nn.sigmoid(gi[:, :_DOUT] + gh[:, :_DOUT])
    z = jax.nn.sigmoid(gi[:, _DOUT:2 * _DOUT] + gh[:, _DOUT:2 * _DOUT])
    n = jnp.tanh(gi[:, 2 * _DOUT:] + r * gh[:, 2 * _DOUT:])
    hn = (1.0 - z) * n + z * h
    out_ref[...] = hn
    res = jnp.dot(hn.astype(jnp.bfloat16), wcat_ref[...],
                  preferred_element_type=jnp.float32) + bcat_ref[...]
    res = res.astype(jnp.bfloat16)
    for e in range(_NET):
        t_ref[e] = res[:, e * _DOUT:(e + 1) * _DOUT]


def _gru_call(aa, h, wih, whh, bih, bhh, wcat, bcat):
    return pl.pallas_call(
        _gru_body,
        grid=(_NB,),
        in_specs=[
            pl.BlockSpec((_NC, _BN, _DOUT), lambda i: (0, i, 0)),
            pl.BlockSpec((_BN, _DOUT), lambda i: (i, 0)),
            pl.BlockSpec((_DOUT, 3 * _DOUT), lambda i: (0, 0)),
            pl.BlockSpec((_DOUT, 3 * _DOUT), lambda i: (0, 0)),
            pl.BlockSpec((1, 3 * _DOUT), lambda i: (0, 0)),
            pl.BlockSpec((1, 3 * _DOUT), lambda i: (0, 0)),
            pl.BlockSpec((_DOUT, _NET * _DOUT), lambda i: (0, 0)),
            pl.BlockSpec((1, _NET * _DOUT), lambda i: (0, 0)),
        ],
        out_specs=[
            pl.BlockSpec((_BN, _DOUT), lambda i: (i, 0)),
            pl.BlockSpec((_NET, _BN, _DOUT), lambda i: (0, i, 0)),
        ],
        out_shape=[
            jax.ShapeDtypeStruct((_N, _DOUT), jnp.float32),
            jax.ShapeDtypeStruct((_NET, _N, _DOUT), jnp.bfloat16),
        ],
    )(aa, h, wih, whh, bih, bhh, wcat, bcat)


def _bn_relu(y, g, b, length):
    s1 = jnp.sum(y, axis=0)
    s2 = jnp.sum(y * y, axis=0)
    mean = s1 / length
    var = s2 / length - mean * mean
    inv = lax.rsqrt(var + 1e-5)
    return jnp.maximum((y - mean[None, :]) * inv[None, :] * g + b, 0.0)


def _pool3(y, length):
    half = (length - 2) // 2
    t = y[:2 * half].reshape(half, 2, y.shape[1])
    m1 = jnp.maximum(t[:, 0, :], t[:, 1, :])
    e2 = y[2:2 + 2 * half].reshape(half, 2, y.shape[1])[:, 0, :]
    return jnp.maximum(m1, e2)


def _pool2(y, length):
    half = length // 2
    t = y[:2 * half].reshape(half, 2, y.shape[1])
    return jnp.maximum(t[:, 0, :], t[:, 1, :])


def _heady_body(h_ref, w1_ref, b1_ref, w2_ref, b2_ref, g_ref, b_ref,
                out_ref):
    x = h_ref[...].astype(jnp.bfloat16)
    y = (jnp.dot(x[0:_N - 2], w1_ref[0], preferred_element_type=jnp.float32)
         + jnp.dot(x[1:_N - 1], w1_ref[1], preferred_element_type=jnp.float32)
         + jnp.dot(x[2:_N], w1_ref[2], preferred_element_type=jnp.float32)
         + b1_ref[...])
    y = _bn_relu(y, g_ref[...], b_ref[...], _N - 2)
    y = _pool3(y, _N - 2)
    y2 = jnp.dot(y.astype(jnp.bfloat16), w2_ref[...],
                 preferred_element_type=jnp.float32) + b2_ref[...]
    y2 = _bn_relu(y2, g_ref[...], b_ref[...], 4998)
    y2 = _pool2(y2, 4998)
    out_ref[0:2499] = y2


def _heady_call(h, w1, b1, w2, b2, g, b):
    return pl.pallas_call(
        _heady_body,
        out_shape=jax.ShapeDtypeStruct((2504, _DOUT), jnp.float32),
    )(h, w1, b1, w2, b2, g, b)


def _headz_body(h_ref, f_ref, wc1h_ref, wc1x_ref, bc1_ref, wc2_ref, bc2_ref,
                g_ref, b_ref, out_ref):
    x = h_ref[...].astype(jnp.bfloat16)
    f = f_ref[...].astype(jnp.bfloat16)
    z = bc1_ref[...]
    for dl in range(3):
        z = z + jnp.dot(x[dl:_N - 2 + dl], wc1h_ref[dl],
                        preferred_element_type=jnp.float32)
        z = z + jnp.dot(f[dl:_N - 2 + dl], wc1x_ref[dl],
                        preferred_element_type=jnp.float32)
    z = _bn_relu(z, g_ref[...], b_ref[...], _N - 2)
    z = _pool3(z, _N - 2)
    z2 = jnp.dot(z.astype(jnp.bfloat16), wc2_ref[...],
                 preferred_element_type=jnp.float32) + bc2_ref[...]
    z2 = _bn_relu(z2, g_ref[...], b_ref[...], 4998)
    z2 = _pool2(z2, 4998)
    out_ref[0:2499] = z2


def _headz_call(h, f, wc1h, wc1x, bc1, wc2, bc2, g, b):
    return pl.pallas_call(
        _headz_body,
        out_shape=jax.ShapeDtypeStruct((2504, _DIN + _DOUT), jnp.float32),
    )(h, f, wc1h, wc1x, bc1, wc2, bc2, g, b)


def _combine_body(y_ref, z_ref, wy_ref, by_ref, wz_ref, bz_ref, out_ref):
    yv = jnp.dot(y_ref[0:2499], wy_ref[...],
                 preferred_element_type=jnp.float32) + by_ref[...]
    zv = jnp.dot(z_ref[0:2499], wz_ref[...],
                 preferred_element_type=jnp.float32) + bz_ref[...]
    avg = jnp.sum(yv * zv, axis=0) / 2499.0
    out_ref[...] = jax.nn.sigmoid(avg)[None, :]


def _combine_call(y2, z2, wy, by, wz, bz):
    return pl.pallas_call(
        _combine_body,
        out_shape=jax.ShapeDtypeStruct((1, 128), jnp.float32),
    )(y2, z2, wy, by, wz, bz)


def kernel(features, edge_index, edge_types, W_et, b_et, W_ih, W_hh, b_ih,
           b_hh, conv1_w, conv1_b, conv2_w, conv2_b, convc1_w, convc1_b,
           convc2_w, convc2_b, bn_g, bn_b, bnc_g, bnc_b, mlp_y_w, mlp_y_b,
           mlp_z_w, mlp_z_b):
    f32 = jnp.float32
    conc = _DIN + _DOUT

    wt = jnp.transpose(W_et, (0, 2, 1))
    wcat = jnp.transpose(wt, (1, 0, 2)).reshape(
        _DOUT, _NET * _DOUT).astype(jnp.bfloat16)
    bcat = b_et.reshape(1, _NET * _DOUT)
    wih = W_ih.T.astype(jnp.bfloat16)
    whh = W_hh.T.astype(jnp.bfloat16)
    bih = b_ih.reshape(1, -1)
    bhh = b_hh.reshape(1, -1)
    w1 = jnp.transpose(conv1_w, (2, 1, 0)).astype(jnp.bfloat16)
    b1 = conv1_b.reshape(1, -1)
    w2 = conv2_w[:, :, 0].T.astype(jnp.bfloat16)
    b2 = conv2_b.reshape(1, -1)
    wc1 = jnp.transpose(convc1_w, (2, 1, 0)).astype(jnp.bfloat16)
    wc1h = wc1[:, :_DOUT, :]
    wc1x = wc1[:, _DOUT:, :]
    bc1 = convc1_b.reshape(1, -1)
    wc2 = convc2_w[:, :, 0].T.astype(jnp.bfloat16)
    bc2 = convc2_b.reshape(1, -1)
    g1 = bn_g.reshape(1, -1)
    bb1 = bn_b.reshape(1, -1)
    gc = bnc_g.reshape(1, -1)
    bbc = bnc_b.reshape(1, -1)
    wy = jnp.zeros((_DOUT, 128), f32).at[:, :2].set(mlp_y_w.T)
    by = jnp.zeros((1, 128), f32).at[:, :2].set(mlp_y_b[None, :])
    wz = jnp.zeros((conc, 128), f32).at[:, :2].set(mlp_z_w.T)
    bz = jnp.zeros((1, 128), f32).at[:, :2].set(mlp_z_b[None, :])

    src = edge_index[0]
    dst = edge_index[1]
    grow = edge_types * _N + src
    npad = _EPS_PAD - _EPS
    pad_g = jnp.broadcast_to(jnp.arange(npad, dtype=jnp.int32),
                             (_NC, _NS, npad))
    pad_d = jnp.broadcast_to(
        _DUMMY + (jnp.arange(npad, dtype=jnp.int32) % (_N_PAD - _N)),
        (_NC, _NS, npad))
    gidx_all = jnp.concatenate(
        [grow.reshape(_NC, _NS, _EPS), pad_g],
        axis=2).reshape(_NC, _NS, _NCHUNK, _CHUNK)
    dp = jnp.concatenate(
        [dst.reshape(_NC, _NS, _EPS), pad_d],
        axis=2).reshape(_NC, _NS, _NCHUNK, _CHUNK)
    zeros_blk = jnp.zeros((_NROWS, 2, 128), jnp.bfloat16)

    h = jnp.concatenate(
        [features, jnp.zeros((_N, _DOUT - _DIN), f32)], axis=1)

    t = _prep_call(h, wcat, bcat)
    for step in range(_STEPS):
        a = _build_sc_segsum()(t.reshape(_NET * _N, 2, 128),
                               gidx_all, dp, zeros_blk)
        h, t = _gru_call(a.reshape(_NC, _N_PAD, _DOUT), h, wih, whh, bih,
                         bhh, wcat, bcat)

    y2 = _heady_call(h, w1, b1, w2, b2, g1, bb1)
    z2 = _headz_call(h, features, wc1h, wc1x, bc1, wc2, bc2, gc, bbc)
    out = _combine_call(y2, z2, wy, by, wz, bz)
    return out[:, :2]


# speedup vs baseline: 1.0544x; 1.0544x over previous
import functools

import jax
import jax.numpy as jnp
from jax import lax
from jax.experimental import pallas as pl
from jax.experimental.pallas import tpu as pltpu
from jax.experimental.pallas import tpu_sc as plsc

_N = 10000
_E = 320000
_DIN = 128
_DOUT = 256
_NET = 4
_STEPS = 8

_NC = 2
_NS = 16
_CHUNK = 64
_EPS = _E // (_NC * _NS)
_NCHUNK = 160
_GROUPS = 5
_GPC = _NCHUNK // _GROUPS
_EPS_PAD = _NCHUNK * _CHUNK
_NROWS = 640
_N_PAD = _NROWS * _NS
_DUMMY = _N

_BN = 400
_NB = _N // _BN


def _sc_segsum_body(t_hbm, gidx_hbm, dst_hbm, zero_hbm, out_hbm,
                    gidx_v, dst_v, rows0_v, rows1_v, rows2_v, rows3_v, gsem, ssem,
                    acc_sh):
    c = lax.axis_index("c")
    s = lax.axis_index("s")
    pltpu.sync_copy(zero_hbm, acc_sh.at[pl.ds(s * _NROWS, _NROWS)])
    plsc.subcore_barrier()

    bufs = [rows0_v, rows1_v, rows2_v, rows3_v]
    nb = len(bufs)
    for grp in range(_GROUPS):
        pltpu.sync_copy(gidx_hbm.at[c, s, pl.ds(grp * _GPC, _GPC)], gidx_v)
        pltpu.sync_copy(dst_hbm.at[c, s, pl.ds(grp * _GPC, _GPC)], dst_v)
        g_desc = [None] * _GPC
        s_desc = [None] * _GPC
        for j in range(nb - 1):
            g_desc[j] = pltpu.async_copy(
                t_hbm.at[gidx_v.at[j]], bufs[j], gsem)
        for j in range(_GPC):
            b = bufs[j % nb]
            if j + nb - 1 < _GPC:
                if j >= 1:
                    s_desc[j - 1].wait()
                g_desc[j + nb - 1] = pltpu.async_copy(
                    t_hbm.at[gidx_v.at[j + nb - 1]], bufs[(j + nb - 1) % nb],
                    gsem)
            g_desc[j].wait()
            s_desc[j] = pltpu.async_copy(
                b, acc_sh.at[dst_v.at[j]], ssem, add=True)
        for j in range(_GPC - nb, _GPC):
            s_desc[j].wait()
    plsc.subcore_barrier()
    pltpu.sync_copy(acc_sh.at[pl.ds(s * _NROWS, _NROWS)],
                    out_hbm.at[c, pl.ds(s * _NROWS, _NROWS)])


@functools.lru_cache(maxsize=None)
def _build_sc_segsum():
    return pl.kernel(
        _sc_segsum_body,
        out_type=jax.ShapeDtypeStruct((_NC, _N_PAD, 2, 128), jnp.bfloat16),
        mesh=plsc.VectorSubcoreMesh(core_axis_name="c", subcore_axis_name="s",
                                    num_cores=_NC, num_subcores=_NS),
        compiler_params=pltpu.CompilerParams(use_tc_tiling_on_sc=False),
        scratch_types=[
            pltpu.VMEM((_GPC, _CHUNK), jnp.int32),
            pltpu.VMEM((_GPC, _CHUNK), jnp.int32),
            pltpu.VMEM((_CHUNK, 2, 128), jnp.bfloat16),
            pltpu.VMEM((_CHUNK, 2, 128), jnp.bfloat16),
            pltpu.VMEM((_CHUNK, 2, 128), jnp.bfloat16),
            pltpu.VMEM((_CHUNK, 2, 128), jnp.bfloat16),
            pltpu.SemaphoreType.DMA,
            pltpu.SemaphoreType.DMA,
            pltpu.VMEM_SHARED((_N_PAD, 2, 128), jnp.bfloat16),
        ],
    )


def _prep_body(h_ref, w_ref, b_ref, t_ref):
    res = jnp.dot(h_ref[...].astype(jnp.bfloat16), w_ref[...],
                  preferred_element_type=jnp.float32) + b_ref[...]
    res = res.astype(jnp.bfloat16)
    for e in range(_NET):
        t_ref[e] = res[:, e * _DOUT:(e + 1) * _DOUT]


def _prep_call(h, wcat, bcat):
    return pl.pallas_call(
        _prep_body,
        grid=(_NB,),
        in_specs=[
            pl.BlockSpec((_BN, _DOUT), lambda i: (i, 0)),
            pl.BlockSpec((_DOUT, _NC * _NET * 128), lambda i: (0, 0)),
            pl.BlockSpec((1, _NC * _NET * 128), lambda i: (0, 0)),
        ],
        out_specs=pl.BlockSpec((_NET, _BN, _DOUT), lambda i: (0, i, 0)),
        out_shape=jax.ShapeDtypeStruct((_NET, _N, _DOUT), jnp.bfloat16),
    )(h, wcat, bcat)


def _gru_body(aa_ref, h_ref, wih_ref, whh_ref, bih_ref, bhh_ref, wcat_ref,
              bcat_ref, out_ref, t_ref):
    h = h_ref[...]
    a = (aa_ref[0] + aa_ref[1]).astype(jnp.bfloat16)
    gi = jnp.dot(a, wih_ref[...], preferred_element_type=jnp.float32) \
        + bih_ref[...]
    gh = jnp.dot(h.astype(jnp.bfloat16), whh_ref[...],
                 preferred_element_type=jnp.float32) + bhh_ref[...]
    r = jax.nn.sigmoid(gi[:, :_DOUT] + gh[:, :_DOUT])
    z = jax.nn.sigmoid(gi[:, _DOUT:2 * _DOUT] + gh[:, _DOUT:2 * _DOUT])
    n = jnp.tanh(gi[:, 2 * _DOUT:] + r * gh[:, 2 * _DOUT:])
    hn = (1.0 - z) * n + z * h
    out_ref[...] = hn
    res = jnp.dot(hn.astype(jnp.bfloat16), wcat_ref[...],
                  preferred_element_type=jnp.float32) + bcat_ref[...]
    res = res.astype(jnp.bfloat16)
    for e in range(_NET):
        t_ref[e] = res[:, e * _DOUT:(e + 1) * _DOUT]


def _gru_call(aa, h, wih, whh, bih, bhh, wcat, bcat):
    return pl.pallas_call(
        _gru_body,
        grid=(_NB,),
        in_specs=[
            pl.BlockSpec((_NC, _BN, _DOUT), lambda i: (0, i, 0)),
            pl.BlockSpec((_BN, _DOUT), lambda i: (i, 0)),
            pl.BlockSpec((_DOUT, 3 * _DOUT), lambda i: (0, 0)),
            pl.BlockSpec((_DOUT, 3 * _DOUT), lambda i: (0, 0)),
            pl.BlockSpec((1, 3 * _DOUT), lambda i: (0, 0)),
            pl.BlockSpec((1, 3 * _DOUT), lambda i: (0, 0)),
            pl.BlockSpec((_DOUT, _NET * _DOUT), lambda i: (0, 0)),
            pl.BlockSpec((1, _NET * _DOUT), lambda i: (0, 0)),
        ],
        out_specs=[
            pl.BlockSpec((_BN, _DOUT), lambda i: (i, 0)),
            pl.BlockSpec((_NET, _BN, _DOUT), lambda i: (0, i, 0)),
        ],
        out_shape=[
            jax.ShapeDtypeStruct((_N, _DOUT), jnp.float32),
            jax.ShapeDtypeStruct((_NET, _N, _DOUT), jnp.bfloat16),
        ],
    )(aa, h, wih, whh, bih, bhh, wcat, bcat)


def _bn_relu(y, g, b, length):
    s1 = jnp.sum(y, axis=0)
    s2 = jnp.sum(y * y, axis=0)
    mean = s1 / length
    var = s2 / length - mean * mean
    inv = lax.rsqrt(var + 1e-5)
    return jnp.maximum((y - mean[None, :]) * inv[None, :] * g + b, 0.0)


def _pool3(y, length):
    half = (length - 2) // 2
    t = y[:2 * half].reshape(half, 2, y.shape[1])
    m1 = jnp.maximum(t[:, 0, :], t[:, 1, :])
    e2 = y[2:2 + 2 * half].reshape(half, 2, y.shape[1])[:, 0, :]
    return jnp.maximum(m1, e2)


def _pool2(y, length):
    half = length // 2
    t = y[:2 * half].reshape(half, 2, y.shape[1])
    return jnp.maximum(t[:, 0, :], t[:, 1, :])


def _heady_body(h_ref, w1_ref, b1_ref, w2_ref, b2_ref, g_ref, b_ref,
                out_ref):
    x = h_ref[...].astype(jnp.bfloat16)
    y = (jnp.dot(x[0:_N - 2], w1_ref[0], preferred_element_type=jnp.float32)
         + jnp.dot(x[1:_N - 1], w1_ref[1], preferred_element_type=jnp.float32)
         + jnp.dot(x[2:_N], w1_ref[2], preferred_element_type=jnp.float32)
         + b1_ref[...])
    y = _bn_relu(y, g_ref[...], b_ref[...], _N - 2)
    y = _pool3(y, _N - 2)
    y2 = jnp.dot(y.astype(jnp.bfloat16), w2_ref[...],
                 preferred_element_type=jnp.float32) + b2_ref[...]
    y2 = _bn_relu(y2, g_ref[...], b_ref[...], 4998)
    y2 = _pool2(y2, 4998)
    out_ref[0:2499] = y2


def _heady_call(h, w1, b1, w2, b2, g, b):
    return pl.pallas_call(
        _heady_body,
        out_shape=jax.ShapeDtypeStruct((2504, _DOUT), jnp.float32),
    )(h, w1, b1, w2, b2, g, b)


def _headz_body(h_ref, f_ref, wc1h_ref, wc1x_ref, bc1_ref, wc2_ref, bc2_ref,
                g_ref, b_ref, out_ref):
    x = h_ref[...].astype(jnp.bfloat16)
    f = f_ref[...].astype(jnp.bfloat16)
    z = bc1_ref[...]
    for dl in range(3):
        z = z + jnp.dot(x[dl:_N - 2 + dl], wc1h_ref[dl],
                        preferred_element_type=jnp.float32)
        z = z + jnp.dot(f[dl:_N - 2 + dl], wc1x_ref[dl],
                        preferred_element_type=jnp.float32)
    z = _bn_relu(z, g_ref[...], b_ref[...], _N - 2)
    z = _pool3(z, _N - 2)
    z2 = jnp.dot(z.astype(jnp.bfloat16), wc2_ref[...],
                 preferred_element_type=jnp.float32) + bc2_ref[...]
    z2 = _bn_relu(z2, g_ref[...], b_ref[...], 4998)
    z2 = _pool2(z2, 4998)
    out_ref[0:2499] = z2


def _headz_call(h, f, wc1h, wc1x, bc1, wc2, bc2, g, b):
    return pl.pallas_call(
        _headz_body,
        out_shape=jax.ShapeDtypeStruct((2504, _DIN + _DOUT), jnp.float32),
    )(h, f, wc1h, wc1x, bc1, wc2, bc2, g, b)


def _combine_body(y_ref, z_ref, wy_ref, by_ref, wz_ref, bz_ref, out_ref):
    yv = jnp.dot(y_ref[0:2499], wy_ref[...],
                 preferred_element_type=jnp.float32) + by_ref[...]
    zv = jnp.dot(z_ref[0:2499], wz_ref[...],
                 preferred_element_type=jnp.float32) + bz_ref[...]
    avg = jnp.sum(yv * zv, axis=0) / 2499.0
    out_ref[...] = jax.nn.sigmoid(avg)[None, :]


def _combine_call(y2, z2, wy, by, wz, bz):
    return pl.pallas_call(
        _combine_body,
        out_shape=jax.ShapeDtypeStruct((1, 128), jnp.float32),
    )(y2, z2, wy, by, wz, bz)


def kernel(features, edge_index, edge_types, W_et, b_et, W_ih, W_hh, b_ih,
           b_hh, conv1_w, conv1_b, conv2_w, conv2_b, convc1_w, convc1_b,
           convc2_w, convc2_b, bn_g, bn_b, bnc_g, bnc_b, mlp_y_w, mlp_y_b,
           mlp_z_w, mlp_z_b):
    f32 = jnp.float32
    conc = _DIN + _DOUT

    wt = jnp.transpose(W_et, (0, 2, 1))
    wcat = jnp.transpose(wt, (1, 0, 2)).reshape(
        _DOUT, _NET * _DOUT).astype(jnp.bfloat16)
    bcat = b_et.reshape(1, _NET * _DOUT)
    wih = W_ih.T.astype(jnp.bfloat16)
    whh = W_hh.T.astype(jnp.bfloat16)
    bih = b_ih.reshape(1, -1)
    bhh = b_hh.reshape(1, -1)
    w1 = jnp.transpose(conv1_w, (2, 1, 0)).astype(jnp.bfloat16)
    b1 = conv1_b.reshape(1, -1)
    w2 = conv2_w[:, :, 0].T.astype(jnp.bfloat16)
    b2 = conv2_b.reshape(1, -1)
    wc1 = jnp.transpose(convc1_w, (2, 1, 0)).astype(jnp.bfloat16)
    wc1h = wc1[:, :_DOUT, :]
    wc1x = wc1[:, _DOUT:, :]
    bc1 = convc1_b.reshape(1, -1)
    wc2 = convc2_w[:, :, 0].T.astype(jnp.bfloat16)
    bc2 = convc2_b.reshape(1, -1)
    g1 = bn_g.reshape(1, -1)
    bb1 = bn_b.reshape(1, -1)
    gc = bnc_g.reshape(1, -1)
    bbc = bnc_b.reshape(1, -1)
    wy = jnp.zeros((_DOUT, 128), f32).at[:, :2].set(mlp_y_w.T)
    by = jnp.zeros((1, 128), f32).at[:, :2].set(mlp_y_b[None, :])
    wz = jnp.zeros((conc, 128), f32).at[:, :2].set(mlp_z_w.T)
    bz = jnp.zeros((1, 128), f32).at[:, :2].set(mlp_z_b[None, :])

    src = edge_index[0]
    dst = edge_index[1]
    grow = edge_types * _N + src
    npad = _EPS_PAD - _EPS
    pad_g = jnp.broadcast_to(jnp.arange(npad, dtype=jnp.int32),
                             (_NC, _NS, npad))
    pad_d = jnp.broadcast_to(
        _DUMMY + (jnp.arange(npad, dtype=jnp.int32) % (_N_PAD - _N)),
        (_NC, _NS, npad))
    gidx_all = jnp.concatenate(
        [grow.reshape(_NC, _NS, _EPS), pad_g],
        axis=2).reshape(_NC, _NS, _NCHUNK, _CHUNK)
    dp = jnp.concatenate(
        [dst.reshape(_NC, _NS, _EPS), pad_d],
        axis=2).reshape(_NC, _NS, _NCHUNK, _CHUNK)
    zeros_blk = jnp.zeros((_NROWS, 2, 128), jnp.bfloat16)

    h = jnp.concatenate(
        [features, jnp.zeros((_N, _DOUT - _DIN), f32)], axis=1)

    t = _prep_call(h, wcat, bcat)
    for step in range(_STEPS):
        a = _build_sc_segsum()(t.reshape(_NET * _N, 2, 128),
                               gidx_all, dp, zeros_blk)
        h, t = _gru_call(a.reshape(_NC, _N_PAD, _DOUT), h, wih, whh, bih,
                         bhh, wcat, bcat)

    y2 = _heady_call(h, w1, b1, w2, b2, g1, bb1)
    z2 = _headz_call(h, features, wc1h, wc1x, bc1, wc2, bc2, gc, bbc)
    out = _combine_call(y2, z2, wy, by, wz, bz)
    return out[:, :2]


# speedup vs baseline: 1.0863x; 1.0302x over previous
import functools

import jax
import jax.numpy as jnp
from jax import lax
from jax.experimental import pallas as pl
from jax.experimental.pallas import tpu as pltpu
from jax.experimental.pallas import tpu_sc as plsc

_N = 10000
_E = 320000
_DIN = 128
_DOUT = 256
_NET = 4
_STEPS = 8

_NC = 2
_NS = 16
_CHUNK = 64
_EPS = _E // (_NC * _NS)
_NCHUNK = 160
_GROUPS = 2
_GPC = _NCHUNK // _GROUPS
_EPS_PAD = _NCHUNK * _CHUNK
_NROWS = 640
_N_PAD = _NROWS * _NS
_DUMMY = _N

_BN = 400
_NB = _N // _BN


def _sc_segsum_body(t_hbm, gidx_hbm, dst_hbm, zero_hbm, out_hbm,
                    gidx_v, dst_v, rows0_v, rows1_v, rows2_v, rows3_v, gsem, ssem,
                    acc_sh):
    c = lax.axis_index("c")
    s = lax.axis_index("s")
    pltpu.sync_copy(zero_hbm, acc_sh.at[pl.ds(s * _NROWS, _NROWS)])
    plsc.subcore_barrier()

    bufs = [rows0_v, rows1_v, rows2_v, rows3_v]
    nb = len(bufs)
    for grp in range(_GROUPS):
        pltpu.sync_copy(gidx_hbm.at[c, s, pl.ds(grp * _GPC, _GPC)], gidx_v)
        pltpu.sync_copy(dst_hbm.at[c, s, pl.ds(grp * _GPC, _GPC)], dst_v)
        g_desc = [None] * _GPC
        s_desc = [None] * _GPC
        for j in range(nb - 1):
            g_desc[j] = pltpu.async_copy(
                t_hbm.at[gidx_v.at[j]], bufs[j], gsem)
        for j in range(_GPC):
            b = bufs[j % nb]
            if j + nb - 1 < _GPC:
                if j >= 1:
                    s_desc[j - 1].wait()
                g_desc[j + nb - 1] = pltpu.async_copy(
                    t_hbm.at[gidx_v.at[j + nb - 1]], bufs[(j + nb - 1) % nb],
                    gsem)
            g_desc[j].wait()
            s_desc[j] = pltpu.async_copy(
                b, acc_sh.at[dst_v.at[j]], ssem, add=True)
        for j in range(_GPC - nb, _GPC):
            s_desc[j].wait()
    plsc.subcore_barrier()
    pltpu.sync_copy(acc_sh.at[pl.ds(s * _NROWS, _NROWS)],
                    out_hbm.at[c, pl.ds(s * _NROWS, _NROWS)])


@functools.lru_cache(maxsize=None)
def _build_sc_segsum():
    return pl.kernel(
        _sc_segsum_body,
        out_type=jax.ShapeDtypeStruct((_NC, _N_PAD, 2, 128), jnp.bfloat16),
        mesh=plsc.VectorSubcoreMesh(core_axis_name="c", subcore_axis_name="s",
                                    num_cores=_NC, num_subcores=_NS),
        compiler_params=pltpu.CompilerParams(use_tc_tiling_on_sc=False),
        scratch_types=[
            pltpu.VMEM((_GPC, _CHUNK), jnp.int32),
            pltpu.VMEM((_GPC, _CHUNK), jnp.int32),
            pltpu.VMEM((_CHUNK, 2, 128), jnp.bfloat16),
            pltpu.VMEM((_CHUNK, 2, 128), jnp.bfloat16),
            pltpu.VMEM((_CHUNK, 2, 128), jnp.bfloat16),
            pltpu.VMEM((_CHUNK, 2, 128), jnp.bfloat16),
            pltpu.SemaphoreType.DMA,
            pltpu.SemaphoreType.DMA,
            pltpu.VMEM_SHARED((_N_PAD, 2, 128), jnp.bfloat16),
        ],
    )


def _prep_body(h_ref, w_ref, b_ref, t_ref):
    res = jnp.dot(h_ref[...].astype(jnp.bfloat16), w_ref[...],
                  preferred_element_type=jnp.float32) + b_ref[...]
    res = res.astype(jnp.bfloat16)
    for e in range(_NET):
        t_ref[e] = res[:, e * _DOUT:(e + 1) * _DOUT]


def _prep_call(h, wcat, bcat):
    return pl.pallas_call(
        _prep_body,
        grid=(_NB,),
        in_specs=[
            pl.BlockSpec((_BN, _DOUT), lambda i: (i, 0)),
            pl.BlockSpec((_DOUT, _NC * _NET * 128), lambda i: (0, 0)),
            pl.BlockSpec((1, _NC * _NET * 128), lambda i: (0, 0)),
        ],
        out_specs=pl.BlockSpec((_NET, _BN, _DOUT), lambda i: (0, i, 0)),
        out_shape=jax.ShapeDtypeStruct((_NET, _N, _DOUT), jnp.bfloat16),
    )(h, wcat, bcat)


def _gru_body(aa_ref, h_ref, wih_ref, whh_ref, bih_ref, bhh_ref, wcat_ref,
              bcat_ref, out_ref, t_ref):
    h = h_ref[...]
    a = (aa_ref[0] + aa_ref[1]).astype(jnp.bfloat16)
    gi = jnp.dot(a, wih_ref[...], preferred_element_type=jnp.float32) \
        + bih_ref[...]
    gh = jnp.dot(h.astype(jnp.bfloat16), whh_ref[...],
                 preferred_element_type=jnp.float32) + bhh_ref[...]
    r = jax.nn.sigmoid(gi[:, :_DOUT] + gh[:, :_DOUT])
    z = jax.nn.sigmoid(gi[:, _DOUT:2 * _DOUT] + gh[:, _DOUT:2 * _DOUT])
    n = jnp.tanh(gi[:, 2 * _DOUT:] + r * gh[:, 2 * _DOUT:])
    hn = (1.0 - z) * n + z * h
    out_ref[...] = hn
    res = jnp.dot(hn.astype(jnp.bfloat16), wcat_ref[...],
                  preferred_element_type=jnp.float32) + bcat_ref[...]
    res = res.astype(jnp.bfloat16)
    for e in range(_NET):
        t_ref[e] = res[:, e * _DOUT:(e + 1) * _DOUT]


def _gru_call(aa, h, wih, whh, bih, bhh, wcat, bcat):
    return pl.pallas_call(
        _gru_body,
        grid=(_NB,),
        in_specs=[
            pl.BlockSpec((_NC, _BN, _DOUT), lambda i: (0, i, 0)),
            pl.BlockSpec((_BN, _DOUT), lambda i: (i, 0)),
            pl.BlockSpec((_DOUT, 3 * _DOUT), lambda i: (0, 0)),
            pl.BlockSpec((_DOUT, 3 * _DOUT), lambda i: (0, 0)),
            pl.BlockSpec((1, 3 * _DOUT), lambda i: (0, 0)),
            pl.BlockSpec((1, 3 * _DOUT), lambda i: (0, 0)),
            pl.BlockSpec((_DOUT, _NET * _DOUT), lambda i: (0, 0)),
            pl.BlockSpec((1, _NET * _DOUT), lambda i: (0, 0)),
        ],
        out_specs=[
            pl.BlockSpec((_BN, _DOUT), lambda i: (i, 0)),
            pl.BlockSpec((_NET, _BN, _DOUT), lambda i: (0, i, 0)),
        ],
        out_shape=[
            jax.ShapeDtypeStruct((_N, _DOUT), jnp.float32),
            jax.ShapeDtypeStruct((_NET, _N, _DOUT), jnp.bfloat16),
        ],
    )(aa, h, wih, whh, bih, bhh, wcat, bcat)


def _bn_relu(y, g, b, length):
    s1 = jnp.sum(y, axis=0)
    s2 = jnp.sum(y * y, axis=0)
    mean = s1 / length
    var = s2 / length - mean * mean
    inv = lax.rsqrt(var + 1e-5)
    return jnp.maximum((y - mean[None, :]) * inv[None, :] * g + b, 0.0)


def _pool3(y, length):
    half = (length - 2) // 2
    t = y[:2 * half].reshape(half, 2, y.shape[1])
    m1 = jnp.maximum(t[:, 0, :], t[:, 1, :])
    e2 = y[2:2 + 2 * half].reshape(half, 2, y.shape[1])[:, 0, :]
    return jnp.maximum(m1, e2)


def _pool2(y, length):
    half = length // 2
    t = y[:2 * half].reshape(half, 2, y.shape[1])
    return jnp.maximum(t[:, 0, :], t[:, 1, :])


def _heady_body(h_ref, w1_ref, b1_ref, w2_ref, b2_ref, g_ref, b_ref,
                out_ref):
    x = h_ref[...]
    y = (jnp.dot(x[0:_N - 2], w1_ref[0], preferred_element_type=jnp.float32)
         + jnp.dot(x[1:_N - 1], w1_ref[1], preferred_element_type=jnp.float32)
         + jnp.dot(x[2:_N], w1_ref[2], preferred_element_type=jnp.float32)
         + b1_ref[...])
    y = _bn_relu(y, g_ref[...], b_ref[...], _N - 2)
    y = _pool3(y, _N - 2)
    y2 = jnp.dot(y, w2_ref[...], preferred_element_type=jnp.float32) \
        + b2_ref[...]
    y2 = _bn_relu(y2, g_ref[...], b_ref[...], 4998)
    y2 = _pool2(y2, 4998)
    out_ref[0:2499] = y2


def _heady_call(h, w1, b1, w2, b2, g, b):
    return pl.pallas_call(
        _heady_body,
        out_shape=jax.ShapeDtypeStruct((2504, _DOUT), jnp.float32),
    )(h, w1, b1, w2, b2, g, b)


def _headz_body(h_ref, f_ref, wc1h_ref, wc1x_ref, bc1_ref, wc2_ref, bc2_ref,
                g_ref, b_ref, out_ref):
    x = h_ref[...]
    f = f_ref[...]
    z = bc1_ref[...]
    for dl in range(3):
        z = z + jnp.dot(x[dl:_N - 2 + dl], wc1h_ref[dl],
                        preferred_element_type=jnp.float32)
        z = z + jnp.dot(f[dl:_N - 2 + dl], wc1x_ref[dl],
                        preferred_element_type=jnp.float32)
    z = _bn_relu(z, g_ref[...], b_ref[...], _N - 2)
    z = _pool3(z, _N - 2)
    z2 = jnp.dot(z, wc2_ref[...], preferred_element_type=jnp.float32) \
        + bc2_ref[...]
    z2 = _bn_relu(z2, g_ref[...], b_ref[...], 4998)
    z2 = _pool2(z2, 4998)
    out_ref[0:2499] = z2


def _headz_call(h, f, wc1h, wc1x, bc1, wc2, bc2, g, b):
    return pl.pallas_call(
        _headz_body,
        out_shape=jax.ShapeDtypeStruct((2504, _DIN + _DOUT), jnp.float32),
    )(h, f, wc1h, wc1x, bc1, wc2, bc2, g, b)


def _combine_body(y_ref, z_ref, wy_ref, by_ref, wz_ref, bz_ref, out_ref):
    yv = jnp.dot(y_ref[0:2499], wy_ref[...],
                 preferred_element_type=jnp.float32) + by_ref[...]
    zv = jnp.dot(z_ref[0:2499], wz_ref[...],
                 preferred_element_type=jnp.float32) + bz_ref[...]
    avg = jnp.sum(yv * zv, axis=0) / 2499.0
    out_ref[...] = jax.nn.sigmoid(avg)[None, :]


def _combine_call(y2, z2, wy, by, wz, bz):
    return pl.pallas_call(
        _combine_body,
        out_shape=jax.ShapeDtypeStruct((1, 128), jnp.float32),
    )(y2, z2, wy, by, wz, bz)


def kernel(features, edge_index, edge_types, W_et, b_et, W_ih, W_hh, b_ih,
           b_hh, conv1_w, conv1_b, conv2_w, conv2_b, convc1_w, convc1_b,
           convc2_w, convc2_b, bn_g, bn_b, bnc_g, bnc_b, mlp_y_w, mlp_y_b,
           mlp_z_w, mlp_z_b):
    f32 = jnp.float32
    conc = _DIN + _DOUT

    wt = jnp.transpose(W_et, (0, 2, 1))
    wcat = jnp.transpose(wt, (1, 0, 2)).reshape(
        _DOUT, _NET * _DOUT).astype(jnp.bfloat16)
    bcat = b_et.reshape(1, _NET * _DOUT)
    wih = W_ih.T.astype(jnp.bfloat16)
    whh = W_hh.T.astype(jnp.bfloat16)
    bih = b_ih.reshape(1, -1)
    bhh = b_hh.reshape(1, -1)
    w1 = jnp.transpose(conv1_w, (2, 1, 0))
    b1 = conv1_b.reshape(1, -1)
    w2 = conv2_w[:, :, 0].T
    b2 = conv2_b.reshape(1, -1)
    wc1 = jnp.transpose(convc1_w, (2, 1, 0))
    wc1h = wc1[:, :_DOUT, :]
    wc1x = wc1[:, _DOUT:, :]
    bc1 = convc1_b.reshape(1, -1)
    wc2 = convc2_w[:, :, 0].T
    bc2 = convc2_b.reshape(1, -1)
    g1 = bn_g.reshape(1, -1)
    bb1 = bn_b.reshape(1, -1)
    gc = bnc_g.reshape(1, -1)
    bbc = bnc_b.reshape(1, -1)
    wy = jnp.zeros((_DOUT, 128), f32).at[:, :2].set(mlp_y_w.T)
    by = jnp.zeros((1, 128), f32).at[:, :2].set(mlp_y_b[None, :])
    wz = jnp.zeros((conc, 128), f32).at[:, :2].set(mlp_z_w.T)
    bz = jnp.zeros((1, 128), f32).at[:, :2].set(mlp_z_b[None, :])

    src = edge_index[0]
    dst = edge_index[1]
    grow = edge_types * _N + src
    npad = _EPS_PAD - _EPS
    pad_g = jnp.broadcast_to(jnp.arange(npad, dtype=jnp.int32),
                             (_NC, _NS, npad))
    pad_d = jnp.broadcast_to(
        _DUMMY + (jnp.arange(npad, dtype=jnp.int32) % (_N_PAD - _N)),
        (_NC, _NS, npad))
    gidx_all = jnp.concatenate(
        [grow.reshape(_NC, _NS, _EPS), pad_g],
        axis=2).reshape(_NC, _NS, _NCHUNK, _CHUNK)
    dp = jnp.concatenate(
        [dst.reshape(_NC, _NS, _EPS), pad_d],
        axis=2).reshape(_NC, _NS, _NCHUNK, _CHUNK)
    zeros_blk = jnp.zeros((_NROWS, 2, 128), jnp.bfloat16)

    h = jnp.concatenate(
        [features, jnp.zeros((_N, _DOUT - _DIN), f32)], axis=1)

    t = _prep_call(h, wcat, bcat)
    for step in range(_STEPS):
        a = _build_sc_segsum()(t.reshape(_NET * _N, 2, 128),
                               gidx_all, dp, zeros_blk)
        h, t = _gru_call(a.reshape(_NC, _N_PAD, _DOUT), h, wih, whh, bih,
                         bhh, wcat, bcat)

    y2 = _heady_call(h, w1, b1, w2, b2, g1, bb1)
    z2 = _headz_call(h, features, wc1h, wc1x, bc1, wc2, bc2, gc, bbc)
    out = _combine_call(y2, z2, wy, by, wz, bz)
    return out[:, :2]


# speedup vs baseline: 1.0917x; 1.0050x over previous
import functools

import jax
import jax.numpy as jnp
from jax import lax
from jax.experimental import pallas as pl
from jax.experimental.pallas import tpu as pltpu
from jax.experimental.pallas import tpu_sc as plsc

_N = 10000
_E = 320000
_DIN = 128
_DOUT = 256
_NET = 4
_STEPS = 8

_NC = 2
_NS = 16
_CHUNK = 64
_EPS = _E // (_NC * _NS)
_NCHUNK = 160
_GROUPS = 2
_GPC = _NCHUNK // _GROUPS
_EPS_PAD = _NCHUNK * _CHUNK
_NROWS = 640
_N_PAD = _NROWS * _NS
_DUMMY = _N

_BN = 400
_NB = _N // _BN


def _sc_segsum_body(t_hbm, gidx_hbm, dst_hbm, zero_hbm, out_hbm,
                    gidx_v, dst_v, rows0_v, rows1_v, rows2_v, rows3_v, gsem, ssem,
                    acc_sh):
    c = lax.axis_index("c")
    s = lax.axis_index("s")
    pf_g = pltpu.async_copy(gidx_hbm.at[c, s, pl.ds(0, _GPC)], gidx_v, gsem)
    pf_d = pltpu.async_copy(dst_hbm.at[c, s, pl.ds(0, _GPC)], dst_v, gsem)
    pltpu.sync_copy(zero_hbm, acc_sh.at[pl.ds(s * _NROWS, _NROWS)])
    plsc.subcore_barrier()
    pf_g.wait()
    pf_d.wait()

    bufs = [rows0_v, rows1_v, rows2_v, rows3_v]
    nb = len(bufs)
    for grp in range(_GROUPS):
        if grp > 0:
            pltpu.sync_copy(gidx_hbm.at[c, s, pl.ds(grp * _GPC, _GPC)],
                            gidx_v)
            pltpu.sync_copy(dst_hbm.at[c, s, pl.ds(grp * _GPC, _GPC)], dst_v)
        g_desc = [None] * _GPC
        s_desc = [None] * _GPC
        for j in range(nb - 1):
            g_desc[j] = pltpu.async_copy(
                t_hbm.at[gidx_v.at[j]], bufs[j], gsem)
        for j in range(_GPC):
            b = bufs[j % nb]
            if j + nb - 1 < _GPC:
                if j >= 1:
                    s_desc[j - 1].wait()
                g_desc[j + nb - 1] = pltpu.async_copy(
                    t_hbm.at[gidx_v.at[j + nb - 1]], bufs[(j + nb - 1) % nb],
                    gsem)
            g_desc[j].wait()
            s_desc[j] = pltpu.async_copy(
                b, acc_sh.at[dst_v.at[j]], ssem, add=True)
        for j in range(_GPC - nb, _GPC):
            s_desc[j].wait()
    plsc.subcore_barrier()
    pltpu.sync_copy(acc_sh.at[pl.ds(s * _NROWS, _NROWS)],
                    out_hbm.at[c, pl.ds(s * _NROWS, _NROWS)])


@functools.lru_cache(maxsize=None)
def _build_sc_segsum():
    return pl.kernel(
        _sc_segsum_body,
        out_type=jax.ShapeDtypeStruct((_NC, _N_PAD, 2, 128), jnp.bfloat16),
        mesh=plsc.VectorSubcoreMesh(core_axis_name="c", subcore_axis_name="s",
                                    num_cores=_NC, num_subcores=_NS),
        compiler_params=pltpu.CompilerParams(use_tc_tiling_on_sc=False),
        scratch_types=[
            pltpu.VMEM((_GPC, _CHUNK), jnp.int32),
            pltpu.VMEM((_GPC, _CHUNK), jnp.int32),
            pltpu.VMEM((_CHUNK, 2, 128), jnp.bfloat16),
            pltpu.VMEM((_CHUNK, 2, 128), jnp.bfloat16),
            pltpu.VMEM((_CHUNK, 2, 128), jnp.bfloat16),
            pltpu.VMEM((_CHUNK, 2, 128), jnp.bfloat16),
            pltpu.SemaphoreType.DMA,
            pltpu.SemaphoreType.DMA,
            pltpu.VMEM_SHARED((_N_PAD, 2, 128), jnp.bfloat16),
        ],
    )


def _prep_body(h_ref, w_ref, b_ref, t_ref):
    res = jnp.dot(h_ref[...].astype(jnp.bfloat16), w_ref[...],
                  preferred_element_type=jnp.float32) + b_ref[...]
    res = res.astype(jnp.bfloat16)
    for e in range(_NET):
        t_ref[e] = res[:, e * _DOUT:(e + 1) * _DOUT]


def _prep_call(h, wcat, bcat):
    return pl.pallas_call(
        _prep_body,
        grid=(_NB,),
        in_specs=[
            pl.BlockSpec((_BN, _DOUT), lambda i: (i, 0)),
            pl.BlockSpec((_DOUT, _NC * _NET * 128), lambda i: (0, 0)),
            pl.BlockSpec((1, _NC * _NET * 128), lambda i: (0, 0)),
        ],
        out_specs=pl.BlockSpec((_NET, _BN, _DOUT), lambda i: (0, i, 0)),
        out_shape=jax.ShapeDtypeStruct((_NET, _N, _DOUT), jnp.bfloat16),
    )(h, wcat, bcat)


def _gru_body(aa_ref, h_ref, wih_ref, whh_ref, bih_ref, bhh_ref, wcat_ref,
              bcat_ref, out_ref, t_ref):
    h = h_ref[...]
    a = (aa_ref[0] + aa_ref[1]).astype(jnp.bfloat16)
    gi = jnp.dot(a, wih_ref[...], preferred_element_type=jnp.float32) \
        + bih_ref[...]
    gh = jnp.dot(h.astype(jnp.bfloat16), whh_ref[...],
                 preferred_element_type=jnp.float32) + bhh_ref[...]
    r = jax.nn.sigmoid(gi[:, :_DOUT] + gh[:, :_DOUT])
    z = jax.nn.sigmoid(gi[:, _DOUT:2 * _DOUT] + gh[:, _DOUT:2 * _DOUT])
    n = jnp.tanh(gi[:, 2 * _DOUT:] + r * gh[:, 2 * _DOUT:])
    hn = (1.0 - z) * n + z * h
    out_ref[...] = hn
    res = jnp.dot(hn.astype(jnp.bfloat16), wcat_ref[...],
                  preferred_element_type=jnp.float32) + bcat_ref[...]
    res = res.astype(jnp.bfloat16)
    for e in range(_NET):
        t_ref[e] = res[:, e * _DOUT:(e + 1) * _DOUT]


def _gru_call(aa, h, wih, whh, bih, bhh, wcat, bcat):
    return pl.pallas_call(
        _gru_body,
        grid=(_NB,),
        in_specs=[
            pl.BlockSpec((_NC, _BN, _DOUT), lambda i: (0, i, 0)),
            pl.BlockSpec((_BN, _DOUT), lambda i: (i, 0)),
            pl.BlockSpec((_DOUT, 3 * _DOUT), lambda i: (0, 0)),
            pl.BlockSpec((_DOUT, 3 * _DOUT), lambda i: (0, 0)),
            pl.BlockSpec((1, 3 * _DOUT), lambda i: (0, 0)),
            pl.BlockSpec((1, 3 * _DOUT), lambda i: (0, 0)),
            pl.BlockSpec((_DOUT, _NET * _DOUT), lambda i: (0, 0)),
            pl.BlockSpec((1, _NET * _DOUT), lambda i: (0, 0)),
        ],
        out_specs=[
            pl.BlockSpec((_BN, _DOUT), lambda i: (i, 0)),
            pl.BlockSpec((_NET, _BN, _DOUT), lambda i: (0, i, 0)),
        ],
        out_shape=[
            jax.ShapeDtypeStruct((_N, _DOUT), jnp.float32),
            jax.ShapeDtypeStruct((_NET, _N, _DOUT), jnp.bfloat16),
        ],
    )(aa, h, wih, whh, bih, bhh, wcat, bcat)


def _bn_relu(y, g, b, length):
    s1 = jnp.sum(y, axis=0)
    s2 = jnp.sum(y * y, axis=0)
    mean = s1 / length
    var = s2 / length - mean * mean
    inv = lax.rsqrt(var + 1e-5)
    return jnp.maximum((y - mean[None, :]) * inv[None, :] * g + b, 0.0)


def _pool3(y, length):
    half = (length - 2) // 2
    t = y[:2 * half].reshape(half, 2, y.shape[1])
    m1 = jnp.maximum(t[:, 0, :], t[:, 1, :])
    e2 = y[2:2 + 2 * half].reshape(half, 2, y.shape[1])[:, 0, :]
    return jnp.maximum(m1, e2)


def _pool2(y, length):
    half = length // 2
    t = y[:2 * half].reshape(half, 2, y.shape[1])
    return jnp.maximum(t[:, 0, :], t[:, 1, :])


def _heady_body(h_ref, w1_ref, b1_ref, w2_ref, b2_ref, g_ref, b_ref,
                out_ref):
    x = h_ref[...]
    y = (jnp.dot(x[0:_N - 2], w1_ref[0], preferred_element_type=jnp.float32)
         + jnp.dot(x[1:_N - 1], w1_ref[1], preferred_element_type=jnp.float32)
         + jnp.dot(x[2:_N], w1_ref[2], preferred_element_type=jnp.float32)
         + b1_ref[...])
    y = _bn_relu(y, g_ref[...], b_ref[...], _N - 2)
    y = _pool3(y, _N - 2)
    y2 = jnp.dot(y, w2_ref[...], preferred_element_type=jnp.float32) \
        + b2_ref[...]
    y2 = _bn_relu(y2, g_ref[...], b_ref[...], 4998)
    y2 = _pool2(y2, 4998)
    out_ref[0:2499] = y2


def _heady_call(h, w1, b1, w2, b2, g, b):
    return pl.pallas_call(
        _heady_body,
        out_shape=jax.ShapeDtypeStruct((2504, _DOUT), jnp.float32),
    )(h, w1, b1, w2, b2, g, b)


def _headz_body(h_ref, f_ref, wc1h_ref, wc1x_ref, bc1_ref, wc2_ref, bc2_ref,
                g_ref, b_ref, out_ref):
    x = h_ref[...]
    f = f_ref[...]
    z = bc1_ref[...]
    for dl in range(3):
        z = z + jnp.dot(x[dl:_N - 2 + dl], wc1h_ref[dl],
                        preferred_element_type=jnp.float32)
        z = z + jnp.dot(f[dl:_N - 2 + dl], wc1x_ref[dl],
                        preferred_element_type=jnp.float32)
    z = _bn_relu(z, g_ref[...], b_ref[...], _N - 2)
    z = _pool3(z, _N - 2)
    z2 = jnp.dot(z, wc2_ref[...], preferred_element_type=jnp.float32) \
        + bc2_ref[...]
    z2 = _bn_relu(z2, g_ref[...], b_ref[...], 4998)
    z2 = _pool2(z2, 4998)
    out_ref[0:2499] = z2


def _headz_call(h, f, wc1h, wc1x, bc1, wc2, bc2, g, b):
    return pl.pallas_call(
        _headz_body,
        out_shape=jax.ShapeDtypeStruct((2504, _DIN + _DOUT), jnp.float32),
    )(h, f, wc1h, wc1x, bc1, wc2, bc2, g, b)


def _combine_body(y_ref, z_ref, wy_ref, by_ref, wz_ref, bz_ref, out_ref):
    yv = jnp.dot(y_ref[0:2499], wy_ref[...],
                 preferred_element_type=jnp.float32) + by_ref[...]
    zv = jnp.dot(z_ref[0:2499], wz_ref[...],
                 preferred_element_type=jnp.float32) + bz_ref[...]
    avg = jnp.sum(yv * zv, axis=0) / 2499.0
    out_ref[...] = jax.nn.sigmoid(avg)[None, :]


def _combine_call(y2, z2, wy, by, wz, bz):
    return pl.pallas_call(
        _combine_body,
        out_shape=jax.ShapeDtypeStruct((1, 128), jnp.float32),
    )(y2, z2, wy, by, wz, bz)


def kernel(features, edge_index, edge_types, W_et, b_et, W_ih, W_hh, b_ih,
           b_hh, conv1_w, conv1_b, conv2_w, conv2_b, convc1_w, convc1_b,
           convc2_w, convc2_b, bn_g, bn_b, bnc_g, bnc_b, mlp_y_w, mlp_y_b,
           mlp_z_w, mlp_z_b):
    f32 = jnp.float32
    conc = _DIN + _DOUT

    wt = jnp.transpose(W_et, (0, 2, 1))
    wcat = jnp.transpose(wt, (1, 0, 2)).reshape(
        _DOUT, _NET * _DOUT).astype(jnp.bfloat16)
    bcat = b_et.reshape(1, _NET * _DOUT)
    wih = W_ih.T.astype(jnp.bfloat16)
    whh = W_hh.T.astype(jnp.bfloat16)
    bih = b_ih.reshape(1, -1)
    bhh = b_hh.reshape(1, -1)
    w1 = jnp.transpose(conv1_w, (2, 1, 0))
    b1 = conv1_b.reshape(1, -1)
    w2 = conv2_w[:, :, 0].T
    b2 = conv2_b.reshape(1, -1)
    wc1 = jnp.transpose(convc1_w, (2, 1, 0))
    wc1h = wc1[:, :_DOUT, :]
    wc1x = wc1[:, _DOUT:, :]
    bc1 = convc1_b.reshape(1, -1)
    wc2 = convc2_w[:, :, 0].T
    bc2 = convc2_b.reshape(1, -1)
    g1 = bn_g.reshape(1, -1)
    bb1 = bn_b.reshape(1, -1)
    gc = bnc_g.reshape(1, -1)
    bbc = bnc_b.reshape(1, -1)
    wy = jnp.zeros((_DOUT, 128), f32).at[:, :2].set(mlp_y_w.T)
    by = jnp.zeros((1, 128), f32).at[:, :2].set(mlp_y_b[None, :])
    wz = jnp.zeros((conc, 128), f32).at[:, :2].set(mlp_z_w.T)
    bz = jnp.zeros((1, 128), f32).at[:, :2].set(mlp_z_b[None, :])

    src = edge_index[0]
    dst = edge_index[1]
    grow = edge_types * _N + src
    npad = _EPS_PAD - _EPS
    pad_g = jnp.broadcast_to(jnp.arange(npad, dtype=jnp.int32),
                             (_NC, _NS, npad))
    pad_d = jnp.broadcast_to(
        _DUMMY + (jnp.arange(npad, dtype=jnp.int32) % (_N_PAD - _N)),
        (_NC, _NS, npad))
    gidx_all = jnp.concatenate(
        [grow.reshape(_NC, _NS, _EPS), pad_g],
        axis=2).reshape(_NC, _NS, _NCHUNK, _CHUNK)
    dp = jnp.concatenate(
        [dst.reshape(_NC, _NS, _EPS), pad_d],
        axis=2).reshape(_NC, _NS, _NCHUNK, _CHUNK)
    zeros_blk = jnp.zeros((_NROWS, 2, 128), jnp.bfloat16)

    h = jnp.concatenate(
        [features, jnp.zeros((_N, _DOUT - _DIN), f32)], axis=1)

    t = _prep_call(h, wcat, bcat)
    for step in range(_STEPS):
        a = _build_sc_segsum()(t.reshape(_NET * _N, 2, 128),
                               gidx_all, dp, zeros_blk)
        h, t = _gru_call(a.reshape(_NC, _N_PAD, _DOUT), h, wih, whh, bih,
                         bhh, wcat, bcat)

    y2 = _heady_call(h, w1, b1, w2, b2, g1, bb1)
    z2 = _headz_call(h, features, wc1h, wc1x, bc1, wc2, bc2, gc, bbc)
    out = _combine_call(y2, z2, wy, by, wz, bz)
    return out[:, :2]
